# Initial kernel scaffold; baseline (speedup 1.0000x reference)
import functools

import jax
import jax.numpy as jnp
from jax import lax
from jax.experimental import pallas as pl
from jax.experimental.pallas import tpu as pltpu
from jax.experimental.pallas import tpu_sc as plsc

B = 16384
HID = 64
TXT_L = 50
FEAT_D = 128

NC, NS = 2, 16
NW = NC * NS
BPW = B // NW
CHUNK = 128
NCK = BPW // CHUNK
LANES = 16


def _make_sc_item_proj():
  mesh = plsc.VectorSubcoreMesh(core_axis_name="c", subcore_axis_name="s",
                                num_cores=NC, num_subcores=NS)

  @functools.partial(
      pl.kernel,
      out_type=jax.ShapeDtypeStruct((B, HID), jnp.float32),
      mesh=mesh,
      scratch_types=[
          pltpu.VMEM((TXT_L, NCK, CHUNK), jnp.int32),
          pltpu.VMEM((NCK, CHUNK), jnp.int32),
          pltpu.VMEM((BPW,), jnp.int32),
          pltpu.VMEM((BPW,), jnp.float32),
          pltpu.VMEM((BPW, HID), jnp.float32),
          pltpu.VMEM((BPW, HID), jnp.float32),
          pltpu.SemaphoreType.DMA,
          pltpu.SemaphoreType.DMA,
      ],
  )
  def sc_item_proj(cat_hbm, txt_hbm, len_hbm, tcat_hbm, ttxt_hbm, out_hbm,
                   idx_t, idx_c, len_v, recip_v, acc, rows, sem_cat, sem_txt):
    wid = lax.axis_index("s") * NC + lax.axis_index("c")
    base = wid * BPW

    pltpu.sync_copy(txt_hbm.at[:, pl.ds(wid * NCK, NCK), :], idx_t)
    pltpu.sync_copy(cat_hbm.at[pl.ds(wid * NCK, NCK), :], idx_c)
    pltpu.sync_copy(len_hbm.at[pl.ds(base, BPW)], len_v)

    cat_descs = [
        pltpu.async_copy(tcat_hbm.at[idx_c.at[c]],
                         rows.at[pl.ds(c * CHUNK, CHUNK)], sem_cat)
        for c in range(NCK)
    ]
    init_descs = [
        pltpu.async_copy(ttxt_hbm.at[idx_t.at[0, c]],
                         acc.at[pl.ds(c * CHUNK, CHUNK)], sem_txt)
        for c in range(NCK)
    ]

    def recip_body(i, carry):
      l16 = len_v[pl.ds(i * LANES, LANES)]
      recip_v[pl.ds(i * LANES, LANES)] = 1.0 / l16.astype(jnp.float32)
      return carry
    lax.fori_loop(0, BPW // LANES, recip_body, 0, unroll=4)

    for d in init_descs:
      d.wait()

    def txt_body(l, carry):
      descs = [
          pltpu.async_copy(ttxt_hbm.at[idx_t.at[l, c]],
                           acc.at[pl.ds(c * CHUNK, CHUNK)], sem_txt, add=True)
          for c in range(NCK)
      ]
      for d in descs:
        d.wait()
      return carry
    lax.fori_loop(1, TXT_L, txt_body, 0)

    for d in cat_descs:
      d.wait()

    def comb_body(b, carry):
      bvec = jnp.full((LANES,), b, jnp.int32)
      r = plsc.load_gather(recip_v, [bvec])
      for c in range(HID // LANES):
        sl = pl.ds(c * LANES, LANES)
        rows[b, sl] = rows[b, sl] + acc[b, sl] * r
      return carry
    lax.fori_loop(0, BPW, comb_body, 0, unroll=2)

    pltpu.sync_copy(rows, out_hbm.at[pl.ds(base, BPW)])

  return sc_item_proj


_sc_item_proj = _make_sc_item_proj()


def _tc_user_proj(user_feat, w_feat, b_feat):
  blk = 2048

  def mm(x_ref, w_ref, b_ref, o_ref):
    o_ref[...] = lax.dot_general(
        x_ref[...], w_ref[...], (((1,), (1,)), ((), ())),
        preferred_element_type=jnp.float32) + b_ref[...]

  return pl.pallas_call(
      mm,
      grid=(B // blk,),
      in_specs=[pl.BlockSpec((blk, FEAT_D), lambda i: (i, 0)),
                pl.BlockSpec((HID, FEAT_D), lambda i: (0, 0)),
                pl.BlockSpec((1, HID), lambda i: (0, 0))],
      out_specs=pl.BlockSpec((blk, HID), lambda i: (i, 0)),
      out_shape=jax.ShapeDtypeStruct((B, HID), jnp.float32),
  )(user_feat, w_feat, b_feat.reshape(1, HID))


def kernel(item_cat, item_text, text_len, user_feat, table_cat, table_text,
           W_feat, b_feat):
  cat_idx = item_cat.astype(jnp.int32).reshape(NW * NCK, CHUNK)
  text_t = item_text.astype(jnp.int32).T.reshape(TXT_L, NW * NCK, CHUNK)
  len_i = text_len.astype(jnp.int32)
  item_proj = _sc_item_proj(cat_idx, text_t, len_i, table_cat, table_text)
  user_proj = _tc_user_proj(user_feat, W_feat, b_feat)
  return jnp.concatenate([item_proj, user_proj], axis=0)


# speedup vs baseline: 5.4219x; 5.4219x over previous
import functools

import jax
import jax.numpy as jnp
from jax import lax
from jax.experimental import pallas as pl
from jax.experimental.pallas import tpu as pltpu
from jax.experimental.pallas import tpu_sc as plsc

B = 16384
HID = 64
TXT_L = 50
FEAT_D = 128

NC, NS = 2, 16
NW = NC * NS
BPW = B // NW
CHUNK = 128
NCK = BPW // CHUNK
LANES = 16


def _make_sc_item_proj():
  mesh = plsc.VectorSubcoreMesh(core_axis_name="c", subcore_axis_name="s",
                                num_cores=NC, num_subcores=NS)

  @functools.partial(
      pl.kernel,
      out_type=jax.ShapeDtypeStruct((B, HID), jnp.float32),
      mesh=mesh,
      compiler_params=pltpu.CompilerParams(use_tc_tiling_on_sc=False),
      scratch_types=[
          pltpu.VMEM((TXT_L, NCK, CHUNK), jnp.int32),
          pltpu.VMEM((NCK, CHUNK), jnp.int32),
          pltpu.VMEM((BPW,), jnp.int32),
          pltpu.VMEM((BPW,), jnp.float32),
          pltpu.VMEM((BPW, HID), jnp.float32),
          pltpu.VMEM((BPW, HID), jnp.float32),
          pltpu.SemaphoreType.DMA,
          pltpu.SemaphoreType.DMA,
      ],
  )
  def sc_item_proj(cat_hbm, txt_hbm, len_hbm, tcat_hbm, ttxt_hbm, out_hbm,
                   idx_t, idx_c, len_v, recip_v, acc, rows, sem_cat, sem_txt):
    wid = lax.axis_index("s") * NC + lax.axis_index("c")
    base = wid * BPW

    pltpu.sync_copy(txt_hbm.at[:, pl.ds(wid * NCK, NCK), :], idx_t)
    pltpu.sync_copy(cat_hbm.at[pl.ds(wid * NCK, NCK), :], idx_c)
    pltpu.sync_copy(len_hbm.at[pl.ds(base, BPW)], len_v)

    cat_descs = [
        pltpu.async_copy(tcat_hbm.at[idx_c.at[c]],
                         rows.at[pl.ds(c * CHUNK, CHUNK)], sem_cat)
        for c in range(NCK)
    ]
    init_descs = [
        pltpu.async_copy(ttxt_hbm.at[idx_t.at[0, c]],
                         acc.at[pl.ds(c * CHUNK, CHUNK)], sem_txt)
        for c in range(NCK)
    ]

    def recip_body(i, carry):
      l16 = len_v[pl.ds(i * LANES, LANES)]
      recip_v[pl.ds(i * LANES, LANES)] = 1.0 / l16.astype(jnp.float32)
      return carry
    lax.fori_loop(0, BPW // LANES, recip_body, 0, unroll=4)

    for d in init_descs:
      d.wait()

    def txt_body(l, carry):
      descs = [
          pltpu.async_copy(ttxt_hbm.at[idx_t.at[l, c]],
                           acc.at[pl.ds(c * CHUNK, CHUNK)], sem_txt, add=True)
          for c in range(NCK)
      ]
      for d in descs:
        d.wait()
      return carry
    lax.fori_loop(1, TXT_L, txt_body, 0)

    for d in cat_descs:
      d.wait()

    def comb_body(g, carry):
      r16 = recip_v[pl.ds(g * LANES, LANES)]
      for j in range(LANES):
        b = g * LANES + j
        rj = lax.gather(
            r16, jnp.full((LANES, 1), j, jnp.int32),
            lax.GatherDimensionNumbers(offset_dims=(),
                                       collapsed_slice_dims=(0,),
                                       start_index_map=(0,)),
            slice_sizes=(1,),
            mode=lax.GatherScatterMode.PROMISE_IN_BOUNDS)
        for c in range(HID // LANES):
          sl = pl.ds(c * LANES, LANES)
          rows[b, sl] = rows[b, sl] + acc[b, sl] * rj
      return carry
    lax.fori_loop(0, BPW // LANES, comb_body, 0)

    pltpu.sync_copy(rows, out_hbm.at[pl.ds(base, BPW)])

  return sc_item_proj


_sc_item_proj = _make_sc_item_proj()


def _tc_user_proj(user_feat, w_feat, b_feat):
  blk = 2048

  def mm(x_ref, w_ref, b_ref, o_ref):
    o_ref[...] = lax.dot_general(
        x_ref[...], w_ref[...], (((1,), (1,)), ((), ())),
        preferred_element_type=jnp.float32) + b_ref[...]

  return pl.pallas_call(
      mm,
      grid=(B // blk,),
      in_specs=[pl.BlockSpec((blk, FEAT_D), lambda i: (i, 0)),
                pl.BlockSpec((HID, FEAT_D), lambda i: (0, 0)),
                pl.BlockSpec((1, HID), lambda i: (0, 0))],
      out_specs=pl.BlockSpec((blk, HID), lambda i: (i, 0)),
      out_shape=jax.ShapeDtypeStruct((B, HID), jnp.float32),
  )(user_feat, w_feat, b_feat.reshape(1, HID))


def kernel(item_cat, item_text, text_len, user_feat, table_cat, table_text,
           W_feat, b_feat):
  cat_idx = item_cat.astype(jnp.int32).reshape(NW * NCK, CHUNK)
  text_t = item_text.astype(jnp.int32).T.reshape(TXT_L, NW * NCK, CHUNK)
  len_i = text_len.astype(jnp.int32)
  item_proj = _sc_item_proj(cat_idx, text_t, len_i, table_cat, table_text)
  user_proj = _tc_user_proj(user_feat, W_feat, b_feat)
  return jnp.concatenate([item_proj, user_proj], axis=0)


# speedup vs baseline: 5.7832x; 1.0666x over previous
import functools

import jax
import jax.numpy as jnp
from jax import lax
from jax.experimental import pallas as pl
from jax.experimental.pallas import tpu as pltpu
from jax.experimental.pallas import tpu_sc as plsc

B = 16384
HID = 64
TXT_L = 50
FEAT_D = 128

NC, NS = 2, 16
NW = NC * NS
BPW = B // NW
CHUNK = 128
NCK = BPW // CHUNK
LAG = 4

_MESH = plsc.VectorSubcoreMesh(core_axis_name="c", subcore_axis_name="s",
                               num_cores=NC, num_subcores=NS)


@functools.partial(
    pl.kernel,
    out_type=jax.ShapeDtypeStruct((B, HID), jnp.float32),
    mesh=_MESH,
    compiler_params=pltpu.CompilerParams(use_tc_tiling_on_sc=False),
    scratch_types=[
        pltpu.VMEM((TXT_L, NCK, CHUNK), jnp.int32),
        pltpu.VMEM((BPW, HID), jnp.float32),
        pltpu.SemaphoreType.DMA,
        pltpu.SemaphoreType.DMA,
    ],
)
def _sc_text(txt_hbm, ttxt_hbm, out_hbm, idx_t, acc, sem_idx, sem_txt):
  wid = lax.axis_index("s") * NC + lax.axis_index("c")
  base = wid * BPW

  pltpu.async_copy(txt_hbm.at[:, pl.ds(wid * NCK, NCK), :], idx_t,
                   sem_idx).wait()

  def fire(l, add):
    return [
        pltpu.async_copy(ttxt_hbm.at[idx_t.at[l, c]],
                         acc.at[pl.ds(c * CHUNK, CHUNK)], sem_txt, add=add)
        for c in range(NCK)
    ]

  for d in fire(0, False):
    d.wait()

  for l in range(1, 1 + LAG):
    fire(l, True)

  def drain():
    for c in range(NCK):
      pltpu.make_async_copy(ttxt_hbm.at[idx_t.at[0, c]],
                            acc.at[pl.ds(c * CHUNK, CHUNK)], sem_txt).wait()

  def txt_body(l, carry):
    fire(l, True)
    drain()
    return carry
  lax.fori_loop(1 + LAG, TXT_L, txt_body, 0)
  for _ in range(LAG):
    drain()

  pltpu.sync_copy(acc, out_hbm.at[pl.ds(base, BPW)])


@functools.partial(
    pl.kernel,
    out_type=jax.ShapeDtypeStruct((B, HID), jnp.float32),
    mesh=_MESH,
    compiler_params=pltpu.CompilerParams(use_tc_tiling_on_sc=False),
    scratch_types=[
        pltpu.VMEM((NCK, CHUNK), jnp.int32),
        pltpu.VMEM((BPW, HID), jnp.float32),
        pltpu.SemaphoreType.DMA,
        pltpu.SemaphoreType.DMA,
    ],
)
def _sc_cat(cat_hbm, tcat_hbm, out_hbm, idx_c, rows, sem_idx, sem_cat):
  wid = lax.axis_index("s") * NC + lax.axis_index("c")
  base = wid * BPW

  pltpu.async_copy(cat_hbm.at[pl.ds(wid * NCK, NCK), :], idx_c,
                   sem_idx).wait()
  descs = [
      pltpu.async_copy(tcat_hbm.at[idx_c.at[c]],
                       rows.at[pl.ds(c * CHUNK, CHUNK)], sem_cat)
      for c in range(NCK)
  ]
  for d in descs:
    d.wait()
  pltpu.sync_copy(rows, out_hbm.at[pl.ds(base, BPW)])


_BLK = 2048
_HALF = B // _BLK


def _tc_combine(text_sum, cat_rows, len_col, user_feat, w_feat, b_feat):
  def body(text_ref, cat_ref, len_ref, x_ref, w_ref, b_ref, o_ref):
    i = pl.program_id(0)

    @pl.when(i < _HALF)
    def _item():
      o_ref[...] = cat_ref[...] + text_ref[...] / len_ref[...]

    @pl.when(i >= _HALF)
    def _user():
      o_ref[...] = lax.dot_general(
          x_ref[...], w_ref[...], (((1,), (1,)), ((), ())),
          preferred_element_type=jnp.float32) + b_ref[...]

  return pl.pallas_call(
      body,
      grid=(2 * _HALF,),
      in_specs=[
          pl.BlockSpec((_BLK, HID), lambda i: (jnp.minimum(i, _HALF - 1), 0)),
          pl.BlockSpec((_BLK, HID), lambda i: (jnp.minimum(i, _HALF - 1), 0)),
          pl.BlockSpec((_BLK, 1), lambda i: (jnp.minimum(i, _HALF - 1), 0)),
          pl.BlockSpec((_BLK, FEAT_D),
                       lambda i: (jnp.maximum(i - _HALF, 0), 0)),
          pl.BlockSpec((HID, FEAT_D), lambda i: (0, 0)),
          pl.BlockSpec((1, HID), lambda i: (0, 0)),
      ],
      out_specs=pl.BlockSpec((_BLK, HID), lambda i: (i, 0)),
      out_shape=jax.ShapeDtypeStruct((2 * B, HID), jnp.float32),
  )(text_sum, cat_rows, len_col, user_feat, w_feat, b_feat.reshape(1, HID))


def kernel(item_cat, item_text, text_len, user_feat, table_cat, table_text,
           W_feat, b_feat):
  cat_idx = item_cat.astype(jnp.int32).reshape(NW * NCK, CHUNK)
  text_t = item_text.astype(jnp.int32).T.reshape(TXT_L, NW * NCK, CHUNK)
  len_col = text_len.astype(jnp.float32).reshape(B, 1)
  text_sum = _sc_text(text_t, table_text)
  cat_rows = _sc_cat(cat_idx, table_cat)
  return _tc_combine(text_sum, cat_rows, len_col, user_feat, W_feat, b_feat)


# speedup vs baseline: 5.8144x; 1.0054x over previous
import functools

import jax
import jax.numpy as jnp
from jax import lax
from jax.experimental import pallas as pl
from jax.experimental.pallas import tpu as pltpu
from jax.experimental.pallas import tpu_sc as plsc

B = 16384
HID = 64
TXT_L = 50
FEAT_D = 128
CAT_V = 1000000

NC, NS = 2, 16
NW = NC * NS
BPW = B // NW
CHUNK = 128
NCK = BPW // CHUNK
LANES = 16

_MESH = plsc.VectorSubcoreMesh(core_axis_name="c", subcore_axis_name="s",
                               num_cores=NC, num_subcores=NS)


@functools.partial(
    pl.kernel,
    out_type=jax.ShapeDtypeStruct((B, HID), jnp.float32),
    mesh=_MESH,
    compiler_params=pltpu.CompilerParams(use_tc_tiling_on_sc=False),
    scratch_types=[
        pltpu.VMEM((TXT_L, NCK, CHUNK), jnp.int32),
        pltpu.VMEM((BPW, HID), jnp.float32),
        pltpu.SemaphoreType.DMA,
        pltpu.SemaphoreType.DMA,
    ],
)
def _sc_text(txt_hbm, ttxt_hbm, out_hbm, idx_t, acc, sem_idx, sem_txt):
  wid = lax.axis_index("s") * NC + lax.axis_index("c")
  base = wid * BPW

  pltpu.async_copy(txt_hbm.at[:, pl.ds(wid * NCK, NCK), :], idx_t,
                   sem_idx).wait()

  def fire(l, add):
    return [
        pltpu.async_copy(ttxt_hbm.at[idx_t.at[l, c]],
                         acc.at[pl.ds(c * CHUNK, CHUNK)], sem_txt, add=add)
        for c in range(NCK)
    ]

  for d in fire(0, False):
    d.wait()

  def txt_body(l, carry):
    for d in fire(l, True):
      d.wait()
    return carry
  lax.fori_loop(1, TXT_L, txt_body, 0)

  pltpu.sync_copy(acc, out_hbm.at[pl.ds(base, BPW)])


@functools.partial(
    pl.kernel,
    out_type=jax.ShapeDtypeStruct((B, 2 * HID), jnp.float32),
    mesh=_MESH,
    scratch_types=[
        pltpu.VMEM((NCK, CHUNK), jnp.int32),
        pltpu.VMEM((NCK, CHUNK), jnp.int32),
        pltpu.VMEM((BPW, 2 * HID), jnp.float32),
        pltpu.SemaphoreType.DMA,
        pltpu.SemaphoreType.DMA,
    ],
)
def _sc_cat(cat_hbm, tcat_hbm, out_hbm, idx_c, pid_c, rows, sem_idx,
            sem_cat):
  wid = lax.axis_index("s") * NC + lax.axis_index("c")
  base = wid * BPW

  pltpu.async_copy(cat_hbm.at[pl.ds(wid * NCK, NCK), :], idx_c,
                   sem_idx).wait()
  for g in range(NCK):
    for k in range(CHUNK // LANES):
      sl = pl.ds(k * LANES, LANES)
      pid_c[g, sl] = lax.shift_right_logical(idx_c[g, sl], 1)

  descs = [
      pltpu.async_copy(tcat_hbm.at[pid_c.at[c]],
                       rows.at[pl.ds(c * CHUNK, CHUNK)], sem_cat)
      for c in range(NCK)
  ]
  for d in descs:
    d.wait()
  pltpu.sync_copy(rows, out_hbm.at[pl.ds(base, BPW)])


_BLK = 2048
_HALF = B // _BLK


def _tc_combine(text_sum, cat_pairs, cat_ids, len_col, user_feat, w_feat,
                b_feat):
  def body(text_ref, pair_ref, ids_ref, len_ref, x_ref, w_ref, b_ref, o_ref):
    i = pl.program_id(0)

    @pl.when(i < _HALF)
    def _item():
      odd = jnp.bitwise_and(ids_ref[...], 1) == 1
      cat = jnp.where(odd, pair_ref[:, HID:], pair_ref[:, :HID])
      o_ref[...] = cat + text_ref[...] / len_ref[...]

    @pl.when(i >= _HALF)
    def _user():
      o_ref[...] = lax.dot_general(
          x_ref[...], w_ref[...], (((1,), (1,)), ((), ())),
          preferred_element_type=jnp.float32) + b_ref[...]

  return pl.pallas_call(
      body,
      grid=(2 * _HALF,),
      in_specs=[
          pl.BlockSpec((_BLK, HID), lambda i: (jnp.minimum(i, _HALF - 1), 0)),
          pl.BlockSpec((_BLK, 2 * HID),
                       lambda i: (jnp.minimum(i, _HALF - 1), 0)),
          pl.BlockSpec((_BLK, 1), lambda i: (jnp.minimum(i, _HALF - 1), 0)),
          pl.BlockSpec((_BLK, 1), lambda i: (jnp.minimum(i, _HALF - 1), 0)),
          pl.BlockSpec((_BLK, FEAT_D),
                       lambda i: (jnp.maximum(i - _HALF, 0), 0)),
          pl.BlockSpec((HID, FEAT_D), lambda i: (0, 0)),
          pl.BlockSpec((1, HID), lambda i: (0, 0)),
      ],
      out_specs=pl.BlockSpec((_BLK, HID), lambda i: (i, 0)),
      out_shape=jax.ShapeDtypeStruct((2 * B, HID), jnp.float32),
  )(text_sum, cat_pairs, cat_ids, len_col, user_feat, w_feat,
    b_feat.reshape(1, HID))


def kernel(item_cat, item_text, text_len, user_feat, table_cat, table_text,
           W_feat, b_feat):
  cat_i32 = item_cat.astype(jnp.int32)
  cat_idx = cat_i32.reshape(NW * NCK, CHUNK)
  text_t = item_text.astype(jnp.int32).T.reshape(TXT_L, NW * NCK, CHUNK)
  len_col = text_len.astype(jnp.float32).reshape(B, 1)
  tcat2 = table_cat.reshape(CAT_V // 2, 2 * HID)
  text_sum = _sc_text(text_t, table_text)
  cat_pairs = _sc_cat(cat_idx, tcat2)
  return _tc_combine(text_sum, cat_pairs, cat_i32.reshape(B, 1), len_col,
                     user_feat, W_feat, b_feat)
